# Initial kernel scaffold; baseline (speedup 1.0000x reference)
import jax
import jax.numpy as jnp
from jax.experimental import pallas as pl


def kernel(x, edge_index, edge_attr, batch, mp_W1, mp_b1, mp_W2, mp_b2, mp_root, mp_bias, dmp_W1, dmp_b1, dmp_W2, dmp_b2, dmp_root, dmp_bias, gru_w_ih, gru_w_hh, gru_b_ih, gru_b_hh, lin_W, lin_b, bn_gamma, bn_beta, sc_W, sc_b):
    raise NotImplementedError("write your pallas kernel here")


# speedup vs baseline: 1.8018x; 1.8018x over previous
import functools

import jax
import jax.numpy as jnp
from jax import lax
from jax.experimental import pallas as pl
from jax.experimental.pallas import tpu as pltpu
from jax.experimental.pallas import tpu_sc as plsc

N = 10000
E = 160000
D = 16
DE = 16
HID = 64
DD = 256
DOUT = 64
W = 128

NTILES = 32
CHUNK = 128
CPT = 40
EPAD = NTILES * CPT * CHUNK
TROWS = 10112
ZR = TROWS // 16

_mesh = plsc.VectorSubcoreMesh(core_axis_name="c", subcore_axis_name="s")


@functools.partial(
    pl.kernel,
    out_type=jax.ShapeDtypeStruct((NTILES * CPT, CHUNK, W), jnp.float32),
    mesh=_mesh,
    scratch_types=[
        pltpu.VMEM((CPT, CHUNK), jnp.int32),
        pltpu.VMEM((CHUNK, W), jnp.float32),
        pltpu.SemaphoreType.DMA,
    ],
)
def _sc_gather(table_hbm, idx_hbm, out_hbm, idx_v, rows_v, sem):
    c = lax.axis_index("c")
    s = lax.axis_index("s")
    wid = s * 2 + c
    pltpu.sync_copy(idx_hbm.at[wid], idx_v)

    def body(j, carry):
        pltpu.async_copy(table_hbm.at[idx_v.at[j]], rows_v, sem).wait()
        pltpu.sync_copy(rows_v, out_hbm.at[wid * CPT + j])
        return carry

    lax.fori_loop(0, CPT, body, 0)


@functools.partial(
    pl.kernel,
    out_type=jax.ShapeDtypeStruct((2, TROWS, W), jnp.float32),
    mesh=_mesh,
    scratch_types=[
        pltpu.VMEM((CPT, CHUNK), jnp.int32),
        pltpu.VMEM((CHUNK, W), jnp.float32),
        pltpu.VMEM_SHARED((TROWS, W), jnp.float32),
        pltpu.SemaphoreType.DMA,
    ],
)
def _sc_scatter(msg_hbm, idx_hbm, zeros_hbm, out_hbm, idx_v, msg_v, table, sem):
    c = lax.axis_index("c")
    s = lax.axis_index("s")
    wid = s * 2 + c
    pltpu.sync_copy(idx_hbm.at[wid], idx_v)
    pltpu.sync_copy(zeros_hbm, table.at[pl.ds(s * ZR, ZR)])
    plsc.subcore_barrier()

    def body(j, carry):
        pltpu.sync_copy(msg_hbm.at[wid * CPT + j], msg_v)
        pltpu.sync_copy(msg_v, table.at[idx_v.at[j]], add=True)
        return carry

    lax.fori_loop(0, CPT, body, 0)
    plsc.subcore_barrier()
    pltpu.sync_copy(table.at[pl.ds(s * ZR, ZR)], out_hbm.at[c, pl.ds(s * ZR, ZR)])


@functools.partial(
    pl.kernel,
    out_type=jax.ShapeDtypeStruct((2, TROWS, W), jnp.float32),
    mesh=_mesh,
    scratch_types=[
        pltpu.VMEM((CPT, CHUNK), jnp.int32),
        pltpu.VMEM((CHUNK, W), jnp.float32),
        pltpu.VMEM_SHARED((TROWS, W), jnp.float32),
    ],
)
def _sc_degree(idx_hbm, ones_hbm, zeros_hbm, out_hbm, idx_v, ones_v, table):
    c = lax.axis_index("c")
    s = lax.axis_index("s")
    wid = s * 2 + c
    pltpu.sync_copy(idx_hbm.at[wid], idx_v)
    pltpu.sync_copy(ones_hbm, ones_v)
    pltpu.sync_copy(zeros_hbm, table.at[pl.ds(s * ZR, ZR)])
    plsc.subcore_barrier()

    def body(j, carry):
        pltpu.sync_copy(ones_v, table.at[idx_v.at[j]], add=True)
        return carry

    lax.fori_loop(0, CPT, body, 0)
    plsc.subcore_barrier()
    pltpu.sync_copy(table.at[pl.ds(s * ZR, ZR)], out_hbm.at[c, pl.ds(s * ZR, ZR)])


def _msg_body(ea_ref, xs_ref, w1_ref, b1_ref, w2_ref, b2_ref, rep_ref, sel_ref, out_ref):
    h = jnp.maximum(jnp.dot(ea_ref[...], w1_ref[...], preferred_element_type=jnp.float32) + b1_ref[...], 0.0)
    w = jnp.dot(h, w2_ref[...], preferred_element_type=jnp.float32) + b2_ref[...]
    xs = xs_ref[:, :D]
    xst = jnp.dot(xs, rep_ref[...], preferred_element_type=jnp.float32)
    msg = jnp.dot(xst * w, sel_ref[...], preferred_element_type=jnp.float32)
    out_ref[...] = jnp.concatenate(
        [msg, jnp.zeros((msg.shape[0], W - D), jnp.float32)], axis=1)


_MSG_BLK = 2048


def _tc_msg(ea_p, xs_flat, w1, b1, w2, b2, rep, sel):
    grid = EPAD // _MSG_BLK
    return pl.pallas_call(
        _msg_body,
        grid=(grid,),
        in_specs=[
            pl.BlockSpec((_MSG_BLK, DE), lambda i: (i, 0)),
            pl.BlockSpec((_MSG_BLK, W), lambda i: (i, 0)),
            pl.BlockSpec((DE, HID), lambda i: (0, 0)),
            pl.BlockSpec((1, HID), lambda i: (0, 0)),
            pl.BlockSpec((HID, DD), lambda i: (0, 0)),
            pl.BlockSpec((1, DD), lambda i: (0, 0)),
            pl.BlockSpec((D, DD), lambda i: (0, 0)),
            pl.BlockSpec((DD, D), lambda i: (0, 0)),
        ],
        out_specs=pl.BlockSpec((_MSG_BLK, W), lambda i: (i, 0)),
        out_shape=jax.ShapeDtypeStruct((EPAD, W), jnp.float32),
    )(ea_p, xs_flat, w1, b1, w2, b2, rep, sel)


def _aggr_body(p_ref, cnt_ref, cur_ref, root_ref, bias_ref, out_ref):
    ssum = p_ref[0, :, :D] + p_ref[1, :, :D]
    cnt = jnp.maximum(cnt_ref[0, :, :D] + cnt_ref[1, :, :D], 1.0)
    m = ssum / cnt + jnp.dot(
        cur_ref[:, :D], root_ref[...], preferred_element_type=jnp.float32) + bias_ref[...]
    out_ref[...] = jnp.concatenate(
        [m, jnp.zeros((m.shape[0], W - D), jnp.float32)], axis=1)


_AGG_BLK = 2000


def _tc_aggr(parts, cnts, cur, root, bias):
    grid = N // _AGG_BLK
    return pl.pallas_call(
        _aggr_body,
        grid=(grid,),
        in_specs=[
            pl.BlockSpec((2, _AGG_BLK, W), lambda i: (0, i, 0)),
            pl.BlockSpec((2, _AGG_BLK, W), lambda i: (0, i, 0)),
            pl.BlockSpec((_AGG_BLK, W), lambda i: (i, 0)),
            pl.BlockSpec((D, D), lambda i: (0, 0)),
            pl.BlockSpec((1, D), lambda i: (0, 0)),
        ],
        out_specs=pl.BlockSpec((_AGG_BLK, W), lambda i: (i, 0)),
        out_shape=jax.ShapeDtypeStruct((N, W), jnp.float32),
    )(parts, cnts, cur, root, bias)


def _aggr_gru_body(p_ref, cnt_ref, m_ref, root_ref, bias_ref, h_ref,
                   wih_ref, whh_ref, bih_ref, bhh_ref, out_ref):
    ssum = p_ref[0, :, :D] + p_ref[1, :, :D]
    cnt = jnp.maximum(cnt_ref[0, :, :D] + cnt_ref[1, :, :D], 1.0)
    m2 = ssum / cnt + jnp.dot(
        m_ref[:, :D], root_ref[...], preferred_element_type=jnp.float32) + bias_ref[...]
    h = h_ref[:, :D]
    gi = jnp.dot(m2, wih_ref[...], preferred_element_type=jnp.float32) + bih_ref[...]
    gh = jnp.dot(h, whh_ref[...], preferred_element_type=jnp.float32) + bhh_ref[...]
    r = jax.nn.sigmoid(gi[:, :D] + gh[:, :D])
    z = jax.nn.sigmoid(gi[:, D:2 * D] + gh[:, D:2 * D])
    nn_ = jnp.tanh(gi[:, 2 * D:] + r * gh[:, 2 * D:])
    hnew = (1.0 - z) * nn_ + z * h
    out_ref[...] = jnp.concatenate(
        [hnew, jnp.zeros((hnew.shape[0], W - D), jnp.float32)], axis=1)


def _tc_aggr_gru(parts, cnts, m, root, bias, h, wihT, whhT, bih, bhh):
    grid = N // _AGG_BLK
    return pl.pallas_call(
        _aggr_gru_body,
        grid=(grid,),
        in_specs=[
            pl.BlockSpec((2, _AGG_BLK, W), lambda i: (0, i, 0)),
            pl.BlockSpec((2, _AGG_BLK, W), lambda i: (0, i, 0)),
            pl.BlockSpec((_AGG_BLK, W), lambda i: (i, 0)),
            pl.BlockSpec((D, D), lambda i: (0, 0)),
            pl.BlockSpec((1, D), lambda i: (0, 0)),
            pl.BlockSpec((_AGG_BLK, W), lambda i: (i, 0)),
            pl.BlockSpec((D, 3 * D), lambda i: (0, 0)),
            pl.BlockSpec((D, 3 * D), lambda i: (0, 0)),
            pl.BlockSpec((1, 3 * D), lambda i: (0, 0)),
            pl.BlockSpec((1, 3 * D), lambda i: (0, 0)),
        ],
        out_specs=pl.BlockSpec((_AGG_BLK, W), lambda i: (i, 0)),
        out_shape=jax.ShapeDtypeStruct((N, W), jnp.float32),
    )(parts, cnts, m, root, bias, h, wihT, whhT, bih, bhh)


def _final_body(cur_ref, linw_ref, linb_ref, g_ref, b_ref, scw_ref, scb_ref, out_ref):
    cur = cur_ref[:, :D]
    y = jnp.dot(cur, linw_ref[...], preferred_element_type=jnp.float32) + linb_ref[...]
    mean = jnp.mean(y, axis=0, keepdims=True)
    var = jnp.mean((y - mean) ** 2, axis=0, keepdims=True)
    yn = (y - mean) * jax.lax.rsqrt(var + 1e-5) * g_ref[...] + b_ref[...]
    out_ref[...] = jnp.maximum(yn, 0.0) + jnp.dot(
        cur, scw_ref[...], preferred_element_type=jnp.float32) + scb_ref[...]


def _tc_final(cur, linw, linb, gamma, beta, scw, scb):
    return pl.pallas_call(
        _final_body,
        out_shape=jax.ShapeDtypeStruct((N, DOUT), jnp.float32),
    )(cur, linw, linb, gamma, beta, scw, scb)


def kernel(x, edge_index, edge_attr, batch, mp_W1, mp_b1, mp_W2, mp_b2,
           mp_root, mp_bias, dmp_W1, dmp_b1, dmp_W2, dmp_b2, dmp_root,
           dmp_bias, gru_w_ih, gru_w_hh, gru_b_ih, gru_b_hh, lin_W, lin_b,
           bn_gamma, bn_beta, sc_W, sc_b):
    src = edge_index[0]
    dst = edge_index[1]
    padn = EPAD - E
    src_p = jnp.concatenate([src, jnp.zeros((padn,), jnp.int32)]).reshape(NTILES, CPT, CHUNK)
    dst_p = jnp.concatenate([dst, jnp.full((padn,), N, jnp.int32)]).reshape(NTILES, CPT, CHUNK)
    ea_p = jnp.concatenate([edge_attr, jnp.zeros((padn, DE), jnp.float32)], axis=0)
    zeros_z = jnp.zeros((ZR, W), jnp.float32)
    ones_b = jnp.ones((CHUNK, W), jnp.float32)
    x_fat = jnp.pad(x, ((0, 0), (0, W - D)))

    iar = jnp.arange(DD, dtype=jnp.int32)
    rep = (iar[None, :] // D == jnp.arange(D, dtype=jnp.int32)[:, None]).astype(jnp.float32)
    sel = (iar[:, None] % D == jnp.arange(D, dtype=jnp.int32)[None, :]).astype(jnp.float32)

    mp_b1r = mp_b1.reshape(1, HID)
    mp_b2r = mp_b2.reshape(1, DD)
    mp_biasr = mp_bias.reshape(1, D)
    dmp_b1r = dmp_b1.reshape(1, HID)
    dmp_b2r = dmp_b2.reshape(1, DD)
    dmp_biasr = dmp_bias.reshape(1, D)
    wihT = gru_w_ih.T
    whhT = gru_w_hh.T
    bihr = gru_b_ih.reshape(1, 3 * D)
    bhhr = gru_b_hh.reshape(1, 3 * D)
    linbr = lin_b.reshape(1, DOUT)
    gammar = bn_gamma.reshape(1, DOUT)
    betar = bn_beta.reshape(1, DOUT)
    scbr = sc_b.reshape(1, DOUT)

    cnts = _sc_degree(dst_p, ones_b, zeros_z)

    h = x_fat
    cur = x_fat
    for _ in range(3):
        xs = _sc_gather(cur, src_p).reshape(EPAD, W)
        msg = _tc_msg(ea_p, xs, mp_W1, mp_b1r, mp_W2, mp_b2r, rep, sel)
        parts = _sc_scatter(msg.reshape(NTILES * CPT, CHUNK, W), dst_p, zeros_z)
        m = _tc_aggr(parts, cnts, cur, mp_root, mp_biasr)

        xs2 = _sc_gather(m, src_p).reshape(EPAD, W)
        msg2 = _tc_msg(ea_p, xs2, dmp_W1, dmp_b1r, dmp_W2, dmp_b2r, rep, sel)
        parts2 = _sc_scatter(msg2.reshape(NTILES * CPT, CHUNK, W), dst_p, zeros_z)
        h = _tc_aggr_gru(parts2, cnts, m, dmp_root, dmp_biasr, h, wihT, whhT, bihr, bhhr)
        cur = h

    return _tc_final(cur, lin_W, linbr, gammar, betar, sc_W, scbr)


# speedup vs baseline: 2.0808x; 1.1549x over previous
import functools

import jax
import jax.numpy as jnp
from jax import lax
from jax.experimental import pallas as pl
from jax.experimental.pallas import tpu as pltpu
from jax.experimental.pallas import tpu_sc as plsc

N = 10000
E = 160000
D = 16
DE = 16
HID = 64
DD = 256
DOUT = 64
W = 128

NTILES = 32
CHUNK = 128
CPT = 40
EPAD = NTILES * CPT * CHUNK
TROWS = 10112
ZR = TROWS // 16

_mesh = plsc.VectorSubcoreMesh(core_axis_name="c", subcore_axis_name="s")


NBG = 5
NB = 2


@functools.partial(
    pl.kernel,
    out_type=jax.ShapeDtypeStruct((NTILES * CPT, CHUNK, W), jnp.float32),
    mesh=_mesh,
    scratch_types=[
        pltpu.VMEM((CPT, CHUNK), jnp.int32),
        [pltpu.VMEM((CHUNK, W), jnp.float32) for _ in range(NBG)],
        [pltpu.SemaphoreType.DMA for _ in range(NBG)],
        [pltpu.SemaphoreType.DMA for _ in range(NBG)],
    ],
)
def _sc_gather(table_hbm, idx_hbm, out_hbm, idx_v, bufs, gsems, wsems):
    c = lax.axis_index("c")
    s = lax.axis_index("s")
    wid = s * 2 + c
    pltpu.sync_copy(idx_hbm.at[wid], idx_v)
    for b in range(NBG):
        pltpu.async_copy(table_hbm.at[idx_v.at[b]], bufs[b], gsems[b])

    def outer(o, carry):
        for b in range(NBG):
            jj = o * NBG + b
            pltpu.make_async_copy(table_hbm.at[idx_v.at[jj]], bufs[b], gsems[b]).wait()
            pltpu.async_copy(bufs[b], out_hbm.at[wid * CPT + jj], wsems[b])

            @pl.when(jj + NBG < CPT)
            def _():
                pltpu.make_async_copy(bufs[b], out_hbm.at[wid * CPT + jj], wsems[b]).wait()
                pltpu.async_copy(table_hbm.at[idx_v.at[jj + NBG]], bufs[b], gsems[b])
        return carry

    lax.fori_loop(0, CPT // NBG, outer, 0)
    for b in range(NBG):
        pltpu.make_async_copy(bufs[b], out_hbm.at[wid * CPT], wsems[b]).wait()


@functools.partial(
    pl.kernel,
    out_type=jax.ShapeDtypeStruct((2, TROWS, W), jnp.float32),
    mesh=_mesh,
    scratch_types=[
        pltpu.VMEM((CPT, CHUNK), jnp.int32),
        [pltpu.VMEM((CHUNK, W), jnp.float32) for _ in range(NB)],
        pltpu.VMEM_SHARED((TROWS, W), jnp.float32),
        [pltpu.SemaphoreType.DMA for _ in range(NB)],
        [pltpu.SemaphoreType.DMA for _ in range(NB)],
    ],
)
def _sc_scatter(msg_hbm, idx_hbm, zeros_hbm, out_hbm, idx_v, bufs, table, lsems, ssems):
    c = lax.axis_index("c")
    s = lax.axis_index("s")
    wid = s * 2 + c
    pltpu.sync_copy(idx_hbm.at[wid], idx_v)
    pltpu.sync_copy(zeros_hbm, table.at[pl.ds(s * ZR, ZR)])
    plsc.subcore_barrier()
    for b in range(NB):
        pltpu.async_copy(msg_hbm.at[wid * CPT + b], bufs[b], lsems[b])

    def outer(o, carry):
        for b in range(NB):
            jj = o * NB + b
            pltpu.make_async_copy(msg_hbm.at[wid * CPT + jj], bufs[b], lsems[b]).wait()
            pltpu.async_copy(bufs[b], table.at[idx_v.at[jj]], ssems[b], add=True)

            @pl.when(jj + NB < CPT)
            def _():
                pltpu.make_async_copy(bufs[b], table.at[idx_v.at[jj]], ssems[b]).wait()
                pltpu.async_copy(msg_hbm.at[wid * CPT + jj + NB], bufs[b], lsems[b])
        return carry

    lax.fori_loop(0, CPT // NB, outer, 0)
    for b in range(NB):
        pltpu.make_async_copy(bufs[b], table.at[idx_v.at[0]], ssems[b]).wait()
    plsc.subcore_barrier()
    pltpu.sync_copy(table.at[pl.ds(s * ZR, ZR)], out_hbm.at[c, pl.ds(s * ZR, ZR)])


@functools.partial(
    pl.kernel,
    out_type=jax.ShapeDtypeStruct((2, TROWS, W), jnp.float32),
    mesh=_mesh,
    scratch_types=[
        pltpu.VMEM((CPT, CHUNK), jnp.int32),
        pltpu.VMEM((CHUNK, W), jnp.float32),
        pltpu.VMEM_SHARED((TROWS, W), jnp.float32),
        pltpu.SemaphoreType.DMA,
    ],
)
def _sc_degree(idx_hbm, ones_hbm, zeros_hbm, out_hbm, idx_v, ones_v, table, sem):
    c = lax.axis_index("c")
    s = lax.axis_index("s")
    wid = s * 2 + c
    pltpu.sync_copy(idx_hbm.at[wid], idx_v)
    pltpu.sync_copy(ones_hbm, ones_v)
    pltpu.sync_copy(zeros_hbm, table.at[pl.ds(s * ZR, ZR)])
    plsc.subcore_barrier()

    def body(j, carry):
        @pl.when(j >= NB)
        def _():
            pltpu.make_async_copy(ones_v, table.at[idx_v.at[0]], sem).wait()
        pltpu.async_copy(ones_v, table.at[idx_v.at[j]], sem, add=True)
        return carry

    lax.fori_loop(0, CPT, body, 0)

    def drain(j, carry):
        pltpu.make_async_copy(ones_v, table.at[idx_v.at[0]], sem).wait()
        return carry

    lax.fori_loop(0, NB, drain, 0)
    plsc.subcore_barrier()
    pltpu.sync_copy(table.at[pl.ds(s * ZR, ZR)], out_hbm.at[c, pl.ds(s * ZR, ZR)])


def _msg_body(ea_ref, xs_ref, w1_ref, b1_ref, w2_ref, b2_ref, rep_ref, sel_ref, out_ref):
    h = jnp.maximum(jnp.dot(ea_ref[...], w1_ref[...], preferred_element_type=jnp.float32) + b1_ref[...], 0.0)
    w = jnp.dot(h, w2_ref[...], preferred_element_type=jnp.float32) + b2_ref[...]
    xs = xs_ref[:, :D]
    xst = jnp.dot(xs, rep_ref[...], preferred_element_type=jnp.float32)
    msg = jnp.dot(xst * w, sel_ref[...], preferred_element_type=jnp.float32)
    out_ref[...] = jnp.concatenate(
        [msg, jnp.zeros((msg.shape[0], W - D), jnp.float32)], axis=1)


_MSG_BLK = 2048


def _tc_msg(ea_p, xs_flat, w1, b1, w2, b2, rep, sel):
    grid = EPAD // _MSG_BLK
    return pl.pallas_call(
        _msg_body,
        grid=(grid,),
        in_specs=[
            pl.BlockSpec((_MSG_BLK, DE), lambda i: (i, 0)),
            pl.BlockSpec((_MSG_BLK, W), lambda i: (i, 0)),
            pl.BlockSpec((DE, HID), lambda i: (0, 0)),
            pl.BlockSpec((1, HID), lambda i: (0, 0)),
            pl.BlockSpec((HID, DD), lambda i: (0, 0)),
            pl.BlockSpec((1, DD), lambda i: (0, 0)),
            pl.BlockSpec((D, DD), lambda i: (0, 0)),
            pl.BlockSpec((DD, D), lambda i: (0, 0)),
        ],
        out_specs=pl.BlockSpec((_MSG_BLK, W), lambda i: (i, 0)),
        out_shape=jax.ShapeDtypeStruct((EPAD, W), jnp.float32),
    )(ea_p, xs_flat, w1, b1, w2, b2, rep, sel)


def _aggr_body(p_ref, cnt_ref, cur_ref, root_ref, bias_ref, out_ref):
    ssum = p_ref[0, :, :D] + p_ref[1, :, :D]
    cnt = jnp.maximum(cnt_ref[0, :, :D] + cnt_ref[1, :, :D], 1.0)
    m = ssum / cnt + jnp.dot(
        cur_ref[:, :D], root_ref[...], preferred_element_type=jnp.float32) + bias_ref[...]
    out_ref[...] = jnp.concatenate(
        [m, jnp.zeros((m.shape[0], W - D), jnp.float32)], axis=1)


_AGG_BLK = 2000


def _tc_aggr(parts, cnts, cur, root, bias):
    grid = N // _AGG_BLK
    return pl.pallas_call(
        _aggr_body,
        grid=(grid,),
        in_specs=[
            pl.BlockSpec((2, _AGG_BLK, W), lambda i: (0, i, 0)),
            pl.BlockSpec((2, _AGG_BLK, W), lambda i: (0, i, 0)),
            pl.BlockSpec((_AGG_BLK, W), lambda i: (i, 0)),
            pl.BlockSpec((D, D), lambda i: (0, 0)),
            pl.BlockSpec((1, D), lambda i: (0, 0)),
        ],
        out_specs=pl.BlockSpec((_AGG_BLK, W), lambda i: (i, 0)),
        out_shape=jax.ShapeDtypeStruct((N, W), jnp.float32),
    )(parts, cnts, cur, root, bias)


def _aggr_gru_body(p_ref, cnt_ref, m_ref, root_ref, bias_ref, h_ref,
                   wih_ref, whh_ref, bih_ref, bhh_ref, out_ref):
    ssum = p_ref[0, :, :D] + p_ref[1, :, :D]
    cnt = jnp.maximum(cnt_ref[0, :, :D] + cnt_ref[1, :, :D], 1.0)
    m2 = ssum / cnt + jnp.dot(
        m_ref[:, :D], root_ref[...], preferred_element_type=jnp.float32) + bias_ref[...]
    h = h_ref[:, :D]
    gi = jnp.dot(m2, wih_ref[...], preferred_element_type=jnp.float32) + bih_ref[...]
    gh = jnp.dot(h, whh_ref[...], preferred_element_type=jnp.float32) + bhh_ref[...]
    r = jax.nn.sigmoid(gi[:, :D] + gh[:, :D])
    z = jax.nn.sigmoid(gi[:, D:2 * D] + gh[:, D:2 * D])
    nn_ = jnp.tanh(gi[:, 2 * D:] + r * gh[:, 2 * D:])
    hnew = (1.0 - z) * nn_ + z * h
    out_ref[...] = jnp.concatenate(
        [hnew, jnp.zeros((hnew.shape[0], W - D), jnp.float32)], axis=1)


def _tc_aggr_gru(parts, cnts, m, root, bias, h, wihT, whhT, bih, bhh):
    grid = N // _AGG_BLK
    return pl.pallas_call(
        _aggr_gru_body,
        grid=(grid,),
        in_specs=[
            pl.BlockSpec((2, _AGG_BLK, W), lambda i: (0, i, 0)),
            pl.BlockSpec((2, _AGG_BLK, W), lambda i: (0, i, 0)),
            pl.BlockSpec((_AGG_BLK, W), lambda i: (i, 0)),
            pl.BlockSpec((D, D), lambda i: (0, 0)),
            pl.BlockSpec((1, D), lambda i: (0, 0)),
            pl.BlockSpec((_AGG_BLK, W), lambda i: (i, 0)),
            pl.BlockSpec((D, 3 * D), lambda i: (0, 0)),
            pl.BlockSpec((D, 3 * D), lambda i: (0, 0)),
            pl.BlockSpec((1, 3 * D), lambda i: (0, 0)),
            pl.BlockSpec((1, 3 * D), lambda i: (0, 0)),
        ],
        out_specs=pl.BlockSpec((_AGG_BLK, W), lambda i: (i, 0)),
        out_shape=jax.ShapeDtypeStruct((N, W), jnp.float32),
    )(parts, cnts, m, root, bias, h, wihT, whhT, bih, bhh)


def _final_body(cur_ref, linw_ref, linb_ref, g_ref, b_ref, scw_ref, scb_ref, out_ref):
    cur = cur_ref[:, :D]
    y = jnp.dot(cur, linw_ref[...], preferred_element_type=jnp.float32) + linb_ref[...]
    mean = jnp.mean(y, axis=0, keepdims=True)
    var = jnp.mean((y - mean) ** 2, axis=0, keepdims=True)
    yn = (y - mean) * jax.lax.rsqrt(var + 1e-5) * g_ref[...] + b_ref[...]
    out_ref[...] = jnp.maximum(yn, 0.0) + jnp.dot(
        cur, scw_ref[...], preferred_element_type=jnp.float32) + scb_ref[...]


def _tc_final(cur, linw, linb, gamma, beta, scw, scb):
    return pl.pallas_call(
        _final_body,
        out_shape=jax.ShapeDtypeStruct((N, DOUT), jnp.float32),
    )(cur, linw, linb, gamma, beta, scw, scb)


def kernel(x, edge_index, edge_attr, batch, mp_W1, mp_b1, mp_W2, mp_b2,
           mp_root, mp_bias, dmp_W1, dmp_b1, dmp_W2, dmp_b2, dmp_root,
           dmp_bias, gru_w_ih, gru_w_hh, gru_b_ih, gru_b_hh, lin_W, lin_b,
           bn_gamma, bn_beta, sc_W, sc_b):
    src = edge_index[0]
    dst = edge_index[1]
    padn = EPAD - E
    src_p = jnp.concatenate([src, jnp.zeros((padn,), jnp.int32)]).reshape(NTILES, CPT, CHUNK)
    dst_p = jnp.concatenate([dst, jnp.full((padn,), N, jnp.int32)]).reshape(NTILES, CPT, CHUNK)
    ea_p = jnp.concatenate([edge_attr, jnp.zeros((padn, DE), jnp.float32)], axis=0)
    zeros_z = jnp.zeros((ZR, W), jnp.float32)
    ones_b = jnp.ones((CHUNK, W), jnp.float32)
    x_fat = jnp.pad(x, ((0, 0), (0, W - D)))

    iar = jnp.arange(DD, dtype=jnp.int32)
    rep = (iar[None, :] // D == jnp.arange(D, dtype=jnp.int32)[:, None]).astype(jnp.float32)
    sel = (iar[:, None] % D == jnp.arange(D, dtype=jnp.int32)[None, :]).astype(jnp.float32)

    mp_b1r = mp_b1.reshape(1, HID)
    mp_b2r = mp_b2.reshape(1, DD)
    mp_biasr = mp_bias.reshape(1, D)
    dmp_b1r = dmp_b1.reshape(1, HID)
    dmp_b2r = dmp_b2.reshape(1, DD)
    dmp_biasr = dmp_bias.reshape(1, D)
    wihT = gru_w_ih.T
    whhT = gru_w_hh.T
    bihr = gru_b_ih.reshape(1, 3 * D)
    bhhr = gru_b_hh.reshape(1, 3 * D)
    linbr = lin_b.reshape(1, DOUT)
    gammar = bn_gamma.reshape(1, DOUT)
    betar = bn_beta.reshape(1, DOUT)
    scbr = sc_b.reshape(1, DOUT)

    cnts = _sc_degree(dst_p, ones_b, zeros_z)

    h = x_fat
    cur = x_fat
    for _ in range(3):
        xs = _sc_gather(cur, src_p).reshape(EPAD, W)
        msg = _tc_msg(ea_p, xs, mp_W1, mp_b1r, mp_W2, mp_b2r, rep, sel)
        parts = _sc_scatter(msg.reshape(NTILES * CPT, CHUNK, W), dst_p, zeros_z)
        m = _tc_aggr(parts, cnts, cur, mp_root, mp_biasr)

        xs2 = _sc_gather(m, src_p).reshape(EPAD, W)
        msg2 = _tc_msg(ea_p, xs2, dmp_W1, dmp_b1r, dmp_W2, dmp_b2r, rep, sel)
        parts2 = _sc_scatter(msg2.reshape(NTILES * CPT, CHUNK, W), dst_p, zeros_z)
        h = _tc_aggr_gru(parts2, cnts, m, dmp_root, dmp_biasr, h, wihT, whhT, bihr, bhhr)
        cur = h

    return _tc_final(cur, lin_W, linbr, gammar, betar, sc_W, scbr)


# speedup vs baseline: 2.1804x; 1.0479x over previous
import functools

import jax
import jax.numpy as jnp
from jax import lax
from jax.experimental import pallas as pl
from jax.experimental.pallas import tpu as pltpu
from jax.experimental.pallas import tpu_sc as plsc

N = 10000
E = 160000
D = 16
DE = 16
HID = 64
DD = 256
DOUT = 64
W = 128

NTILES = 32
CHUNK = 128
CPT = 40
EPAD = NTILES * CPT * CHUNK
TROWS = 10112
ZR = TROWS // 16

_mesh = plsc.VectorSubcoreMesh(core_axis_name="c", subcore_axis_name="s")


NBG = 5
NB = 2


@functools.partial(
    pl.kernel,
    out_type=jax.ShapeDtypeStruct((NTILES * CPT, CHUNK, W), jnp.float32),
    mesh=_mesh,
    scratch_types=[
        pltpu.VMEM((CPT, CHUNK), jnp.int32),
        [pltpu.VMEM((CHUNK, W), jnp.float32) for _ in range(NBG)],
        [pltpu.SemaphoreType.DMA for _ in range(NBG)],
        [pltpu.SemaphoreType.DMA for _ in range(NBG)],
    ],
)
def _sc_gather(table_hbm, idx_hbm, out_hbm, idx_v, bufs, gsems, wsems):
    c = lax.axis_index("c")
    s = lax.axis_index("s")
    wid = s * 2 + c
    pltpu.sync_copy(idx_hbm.at[wid], idx_v)
    for b in range(NBG):
        pltpu.async_copy(table_hbm.at[idx_v.at[b]], bufs[b], gsems[b])

    def outer(o, carry):
        for b in range(NBG):
            jj = o * NBG + b
            pltpu.make_async_copy(table_hbm.at[idx_v.at[jj]], bufs[b], gsems[b]).wait()
            pltpu.async_copy(bufs[b], out_hbm.at[wid * CPT + jj], wsems[b])

            @pl.when(jj + NBG < CPT)
            def _():
                pltpu.make_async_copy(bufs[b], out_hbm.at[wid * CPT + jj], wsems[b]).wait()
                pltpu.async_copy(table_hbm.at[idx_v.at[jj + NBG]], bufs[b], gsems[b])
        return carry

    lax.fori_loop(0, CPT // NBG, outer, 0)
    for b in range(NBG):
        pltpu.make_async_copy(bufs[b], out_hbm.at[wid * CPT], wsems[b]).wait()


@functools.partial(
    pl.kernel,
    out_type=jax.ShapeDtypeStruct((2, TROWS, W), jnp.float32),
    mesh=_mesh,
    scratch_types=[
        pltpu.VMEM((CPT, CHUNK), jnp.int32),
        [pltpu.VMEM((CHUNK, W), jnp.float32) for _ in range(NB)],
        pltpu.VMEM_SHARED((TROWS, W), jnp.float32),
        [pltpu.SemaphoreType.DMA for _ in range(NB)],
        [pltpu.SemaphoreType.DMA for _ in range(NB)],
    ],
)
def _sc_scatter(msg_hbm, idx_hbm, zeros_hbm, out_hbm, idx_v, bufs, table, lsems, ssems):
    c = lax.axis_index("c")
    s = lax.axis_index("s")
    wid = s * 2 + c
    pltpu.sync_copy(idx_hbm.at[wid], idx_v)
    pltpu.sync_copy(zeros_hbm, table.at[pl.ds(s * ZR, ZR)])
    plsc.subcore_barrier()
    for b in range(NB):
        pltpu.async_copy(msg_hbm.at[wid * CPT + b], bufs[b], lsems[b])

    def outer(o, carry):
        for b in range(NB):
            jj = o * NB + b
            pltpu.make_async_copy(msg_hbm.at[wid * CPT + jj], bufs[b], lsems[b]).wait()
            pltpu.async_copy(bufs[b], table.at[idx_v.at[jj]], ssems[b], add=True)

            @pl.when(jj + NB < CPT)
            def _():
                pltpu.make_async_copy(bufs[b], table.at[idx_v.at[jj]], ssems[b]).wait()
                pltpu.async_copy(msg_hbm.at[wid * CPT + jj + NB], bufs[b], lsems[b])
        return carry

    lax.fori_loop(0, CPT // NB, outer, 0)
    for b in range(NB):
        pltpu.make_async_copy(bufs[b], table.at[idx_v.at[0]], ssems[b]).wait()
    plsc.subcore_barrier()
    pltpu.sync_copy(table.at[pl.ds(s * ZR, ZR)], out_hbm.at[c, pl.ds(s * ZR, ZR)])


def _msg_body(ea_ref, xs_ref, w1_ref, b1_ref, w2_ref, b2_ref, rep_ref, sel_ref, out_ref):
    h = jnp.maximum(jnp.dot(ea_ref[...], w1_ref[...], preferred_element_type=jnp.float32) + b1_ref[...], 0.0)
    w = jnp.dot(h, w2_ref[...], preferred_element_type=jnp.float32) + b2_ref[...]
    xs = xs_ref[:, :D]
    xst = jnp.dot(xs, rep_ref[...], preferred_element_type=jnp.float32)
    msg = jnp.dot(xst * w, sel_ref[...], preferred_element_type=jnp.float32)
    out_ref[...] = jnp.concatenate(
        [msg, jnp.ones((msg.shape[0], 1), jnp.float32),
         jnp.zeros((msg.shape[0], W - D - 1), jnp.float32)], axis=1)


_MSG_BLK = 2048


def _tc_msg(ea_p, xs_flat, w1, b1, w2, b2, rep, sel):
    grid = EPAD // _MSG_BLK
    return pl.pallas_call(
        _msg_body,
        grid=(grid,),
        in_specs=[
            pl.BlockSpec((_MSG_BLK, DE), lambda i: (i, 0)),
            pl.BlockSpec((_MSG_BLK, W), lambda i: (i, 0)),
            pl.BlockSpec((DE, HID), lambda i: (0, 0)),
            pl.BlockSpec((1, HID), lambda i: (0, 0)),
            pl.BlockSpec((HID, DD), lambda i: (0, 0)),
            pl.BlockSpec((1, DD), lambda i: (0, 0)),
            pl.BlockSpec((D, DD), lambda i: (0, 0)),
            pl.BlockSpec((DD, D), lambda i: (0, 0)),
        ],
        out_specs=pl.BlockSpec((_MSG_BLK, W), lambda i: (i, 0)),
        out_shape=jax.ShapeDtypeStruct((EPAD, W), jnp.float32),
    )(ea_p, xs_flat, w1, b1, w2, b2, rep, sel)


def _aggr_body(p_ref, cur_ref, root_ref, bias_ref, out_ref):
    ssum = p_ref[0, :, :D] + p_ref[1, :, :D]
    cnt = jnp.maximum(p_ref[0, :, D:D + 1] + p_ref[1, :, D:D + 1], 1.0)
    m = ssum / cnt + jnp.dot(
        cur_ref[:, :D], root_ref[...], preferred_element_type=jnp.float32) + bias_ref[...]
    out_ref[...] = jnp.concatenate(
        [m, jnp.zeros((m.shape[0], W - D), jnp.float32)], axis=1)


_AGG_BLK = 2000


def _tc_aggr(parts, cur, root, bias):
    grid = N // _AGG_BLK
    return pl.pallas_call(
        _aggr_body,
        grid=(grid,),
        in_specs=[
            pl.BlockSpec((2, _AGG_BLK, W), lambda i: (0, i, 0)),
            pl.BlockSpec((_AGG_BLK, W), lambda i: (i, 0)),
            pl.BlockSpec((D, D), lambda i: (0, 0)),
            pl.BlockSpec((1, D), lambda i: (0, 0)),
        ],
        out_specs=pl.BlockSpec((_AGG_BLK, W), lambda i: (i, 0)),
        out_shape=jax.ShapeDtypeStruct((N, W), jnp.float32),
    )(parts, cur, root, bias)


def _aggr_gru_body(p_ref, m_ref, root_ref, bias_ref, h_ref,
                   wih_ref, whh_ref, bih_ref, bhh_ref, out_ref):
    ssum = p_ref[0, :, :D] + p_ref[1, :, :D]
    cnt = jnp.maximum(p_ref[0, :, D:D + 1] + p_ref[1, :, D:D + 1], 1.0)
    m2 = ssum / cnt + jnp.dot(
        m_ref[:, :D], root_ref[...], preferred_element_type=jnp.float32) + bias_ref[...]
    h = h_ref[:, :D]
    gi = jnp.dot(m2, wih_ref[...], preferred_element_type=jnp.float32) + bih_ref[...]
    gh = jnp.dot(h, whh_ref[...], preferred_element_type=jnp.float32) + bhh_ref[...]
    r = jax.nn.sigmoid(gi[:, :D] + gh[:, :D])
    z = jax.nn.sigmoid(gi[:, D:2 * D] + gh[:, D:2 * D])
    nn_ = jnp.tanh(gi[:, 2 * D:] + r * gh[:, 2 * D:])
    hnew = (1.0 - z) * nn_ + z * h
    out_ref[...] = jnp.concatenate(
        [hnew, jnp.zeros((hnew.shape[0], W - D), jnp.float32)], axis=1)


def _tc_aggr_gru(parts, m, root, bias, h, wihT, whhT, bih, bhh):
    grid = N // _AGG_BLK
    return pl.pallas_call(
        _aggr_gru_body,
        grid=(grid,),
        in_specs=[
            pl.BlockSpec((2, _AGG_BLK, W), lambda i: (0, i, 0)),
            pl.BlockSpec((_AGG_BLK, W), lambda i: (i, 0)),
            pl.BlockSpec((D, D), lambda i: (0, 0)),
            pl.BlockSpec((1, D), lambda i: (0, 0)),
            pl.BlockSpec((_AGG_BLK, W), lambda i: (i, 0)),
            pl.BlockSpec((D, 3 * D), lambda i: (0, 0)),
            pl.BlockSpec((D, 3 * D), lambda i: (0, 0)),
            pl.BlockSpec((1, 3 * D), lambda i: (0, 0)),
            pl.BlockSpec((1, 3 * D), lambda i: (0, 0)),
        ],
        out_specs=pl.BlockSpec((_AGG_BLK, W), lambda i: (i, 0)),
        out_shape=jax.ShapeDtypeStruct((N, W), jnp.float32),
    )(parts, m, root, bias, h, wihT, whhT, bih, bhh)


def _final_body(cur_ref, linw_ref, linb_ref, g_ref, b_ref, scw_ref, scb_ref, out_ref):
    cur = cur_ref[:, :D]
    y = jnp.dot(cur, linw_ref[...], preferred_element_type=jnp.float32) + linb_ref[...]
    mean = jnp.mean(y, axis=0, keepdims=True)
    var = jnp.mean((y - mean) ** 2, axis=0, keepdims=True)
    yn = (y - mean) * jax.lax.rsqrt(var + 1e-5) * g_ref[...] + b_ref[...]
    out_ref[...] = jnp.maximum(yn, 0.0) + jnp.dot(
        cur, scw_ref[...], preferred_element_type=jnp.float32) + scb_ref[...]


def _tc_final(cur, linw, linb, gamma, beta, scw, scb):
    return pl.pallas_call(
        _final_body,
        out_shape=jax.ShapeDtypeStruct((N, DOUT), jnp.float32),
    )(cur, linw, linb, gamma, beta, scw, scb)


def kernel(x, edge_index, edge_attr, batch, mp_W1, mp_b1, mp_W2, mp_b2,
           mp_root, mp_bias, dmp_W1, dmp_b1, dmp_W2, dmp_b2, dmp_root,
           dmp_bias, gru_w_ih, gru_w_hh, gru_b_ih, gru_b_hh, lin_W, lin_b,
           bn_gamma, bn_beta, sc_W, sc_b):
    src = edge_index[0]
    dst = edge_index[1]
    padn = EPAD - E
    src_p = jnp.concatenate([src, jnp.zeros((padn,), jnp.int32)]).reshape(NTILES, CPT, CHUNK)
    dst_p = jnp.concatenate([dst, jnp.full((padn,), N, jnp.int32)]).reshape(NTILES, CPT, CHUNK)
    ea_p = jnp.concatenate([edge_attr, jnp.zeros((padn, DE), jnp.float32)], axis=0)
    zeros_z = jnp.zeros((ZR, W), jnp.float32)
    x_fat = jnp.pad(x, ((0, 0), (0, W - D)))

    iar = jnp.arange(DD, dtype=jnp.int32)
    rep = (iar[None, :] // D == jnp.arange(D, dtype=jnp.int32)[:, None]).astype(jnp.float32)
    sel = (iar[:, None] % D == jnp.arange(D, dtype=jnp.int32)[None, :]).astype(jnp.float32)

    mp_b1r = mp_b1.reshape(1, HID)
    mp_b2r = mp_b2.reshape(1, DD)
    mp_biasr = mp_bias.reshape(1, D)
    dmp_b1r = dmp_b1.reshape(1, HID)
    dmp_b2r = dmp_b2.reshape(1, DD)
    dmp_biasr = dmp_bias.reshape(1, D)
    wihT = gru_w_ih.T
    whhT = gru_w_hh.T
    bihr = gru_b_ih.reshape(1, 3 * D)
    bhhr = gru_b_hh.reshape(1, 3 * D)
    linbr = lin_b.reshape(1, DOUT)
    gammar = bn_gamma.reshape(1, DOUT)
    betar = bn_beta.reshape(1, DOUT)
    scbr = sc_b.reshape(1, DOUT)

    h = x_fat
    cur = x_fat
    for _ in range(3):
        xs = _sc_gather(cur, src_p).reshape(EPAD, W)
        msg = _tc_msg(ea_p, xs, mp_W1, mp_b1r, mp_W2, mp_b2r, rep, sel)
        parts = _sc_scatter(msg.reshape(NTILES * CPT, CHUNK, W), dst_p, zeros_z)
        m = _tc_aggr(parts, cur, mp_root, mp_biasr)

        xs2 = _sc_gather(m, src_p).reshape(EPAD, W)
        msg2 = _tc_msg(ea_p, xs2, dmp_W1, dmp_b1r, dmp_W2, dmp_b2r, rep, sel)
        parts2 = _sc_scatter(msg2.reshape(NTILES * CPT, CHUNK, W), dst_p, zeros_z)
        h = _tc_aggr_gru(parts2, m, dmp_root, dmp_biasr, h, wihT, whhT, bihr, bhhr)
        cur = h

    return _tc_final(cur, lin_W, linbr, gammar, betar, sc_W, scbr)
